# baseline (speedup 1.0000x reference)
import functools

import jax
import jax.numpy as jnp
from jax import lax
from jax.experimental import pallas as pl
from jax.experimental.pallas import tpu as pltpu
from jax.experimental.pallas import tpu_sc as plsc

K = 100000
NQ = 16
D = 64
G = 8
DP = G * D
KB8 = 632
NBT = 8
NW = 32
SC_BASE = 40000
PER_W = (K - SC_BASE) // NW
NCH = 3
CHR = PER_W // NCH
BIGF = 3.0e38


def _tc_dist_body(mem_ref, q_ref, bv_ref, bi_ref,
                  wq_scr, wn_scr, bestv_scr, bidx_scr):
    i = pl.program_id(0)

    @pl.when(i == 0)
    def _build_weights():
        q = q_ref[...]
        ident = (jax.lax.broadcasted_iota(jnp.int32, (D, D), 0) ==
                 jax.lax.broadcasted_iota(jnp.int32, (D, D), 1)
                 ).astype(jnp.float32)
        qt = jax.lax.dot_general(
            ident, q, (((1,), (1,)), ((), ())),
            preferred_element_type=jnp.float32,
            precision=jax.lax.Precision.HIGHEST)
        qt_tall = jnp.concatenate([qt] * G, axis=0)
        qt_wide = jnp.concatenate([qt_tall] * G, axis=1)
        rgrp = jax.lax.broadcasted_iota(jnp.int32, (DP, G * NQ), 0) // D
        cgrp = jax.lax.broadcasted_iota(jnp.int32, (DP, G * NQ), 1) // NQ
        gmask = (rgrp == cgrp).astype(jnp.float32)
        wq_scr[...] = -2.0 * qt_wide * gmask
        wn_scr[...] = gmask
        bestv_scr[...] = jnp.full((1, G * NQ), BIGF, jnp.float32)
        bidx_scr[...] = jnp.full((1, G * NQ), K, jnp.int32)

    mem = mem_ref[...]
    dist = (
        jax.lax.dot_general(
            mem, wq_scr[...], (((1,), (0,)), ((), ())),
            preferred_element_type=jnp.float32,
            precision=jax.lax.Precision.HIGHEST)
        + jax.lax.dot_general(
            mem * mem, wn_scr[...], (((1,), (0,)), ((), ())),
            preferred_element_type=jnp.float32,
            precision=jax.lax.Precision.HIGHEST)
    )

    kidx = ((jax.lax.broadcasted_iota(jnp.int32, (KB8, G * NQ), 0)
             + i * KB8) * G
            + jax.lax.broadcasted_iota(jnp.int32, (KB8, G * NQ), 1) // NQ)
    bmin = jnp.min(dist, axis=0, keepdims=True)
    bidx = jnp.min(jnp.where(dist == bmin, kidx, K),
                   axis=0, keepdims=True)
    prev = bestv_scr[...]
    upd = bmin < prev
    bestv_scr[...] = jnp.where(upd, bmin, prev)
    bidx_scr[...] = jnp.where(upd, bidx, bidx_scr[...])

    @pl.when(i == NBT - 1)
    def _final():
        v = bestv_scr[...]
        ix = bidx_scr[...]
        for s in (4 * NQ, 2 * NQ, NQ):
            vs = jnp.concatenate(
                [v[:, s:], jnp.full((1, s), BIGF, jnp.float32)], axis=1)
            ixs = jnp.concatenate(
                [ix[:, s:], jnp.full((1, s), K, jnp.int32)], axis=1)
            take = (vs < v) | ((vs == v) & (ixs < ix))
            v = jnp.where(take, vs, v)
            ix = jnp.where(take, ixs, ix)
        bv_ref[...] = v[:, :NQ]
        bi_ref[...] = ix[:, :NQ]


def _tc_dist(mem_packed, inpt):
    return pl.pallas_call(
        _tc_dist_body,
        grid=(NBT,),
        in_specs=[
            pl.BlockSpec((KB8, DP), lambda i: (i, 0)),
            pl.BlockSpec((NQ, D), lambda i: (0, 0)),
        ],
        out_specs=[
            pl.BlockSpec((1, NQ), lambda i: (0, 0)),
            pl.BlockSpec((1, NQ), lambda i: (0, 0)),
        ],
        out_shape=[
            jax.ShapeDtypeStruct((1, NQ), jnp.float32),
            jax.ShapeDtypeStruct((1, NQ), jnp.int32),
        ],
        scratch_shapes=[
            pltpu.VMEM((DP, G * NQ), jnp.float32),
            pltpu.VMEM((DP, G * NQ), jnp.float32),
            pltpu.VMEM((1, G * NQ), jnp.float32),
            pltpu.VMEM((1, G * NQ), jnp.int32),
        ],
        compiler_params=pltpu.CompilerParams(
            dimension_semantics=("arbitrary",)),
    )(mem_packed, inpt)


def _rot(v, s):
    idx = (jnp.arange(16, dtype=jnp.int32) + s) % 16
    dnums = lax.GatherDimensionNumbers(
        offset_dims=(), collapsed_slice_dims=(0,), start_index_map=(0,))
    return lax.gather(v, idx[:, None], dnums, (1,),
                      mode=lax.GatherScatterMode.PROMISE_IN_BOUNDS)


def _allsum(v):
    for s in (8, 4, 2, 1):
        v = v + _rot(v, s)
    return v


@functools.cache
def _make_sc_dist():
    mesh = plsc.VectorSubcoreMesh(core_axis_name="c", subcore_axis_name="s")

    @functools.partial(
        pl.kernel,
        mesh=mesh,
        out_type=[
            jax.ShapeDtypeStruct((NW, NQ), jnp.float32),
            jax.ShapeDtypeStruct((NW, NQ), jnp.int32),
        ],
        scratch_types=[
            pltpu.VMEM((NQ, D), jnp.float32),
            pltpu.VMEM((CHR, D), jnp.float32),
            pltpu.VMEM((CHR, D), jnp.float32),
            pltpu.VMEM((NQ,), jnp.float32),
            pltpu.VMEM((NQ,), jnp.int32),
            pltpu.SemaphoreType.DMA,
            pltpu.SemaphoreType.DMA,
        ],
        compiler_params=pltpu.CompilerParams(use_tc_tiling_on_sc=False),
    )
    def _sc_dist(q_hbm, table_hbm, bv_hbm, bi_hbm,
                 q_v, buf0, buf1, resv_v, resi_v, sem0, sem1):
        wid = lax.axis_index("s") * 2 + lax.axis_index("c")
        base = SC_BASE + wid * PER_W
        pltpu.sync_copy(q_hbm, q_v)
        bufs = (buf0, buf1)
        sems = (sem0, sem1)
        lanes = jnp.arange(16, dtype=jnp.int32)

        copies = [None] * NCH
        copies[0] = pltpu.async_copy(
            table_hbm.at[pl.ds(base, CHR)], buf0, sem0)

        best_v = jnp.full((NQ,), BIGF, jnp.float32)
        best_i = jnp.full((NQ,), K, jnp.int32)

        for c in range(NCH):
            if c + 1 < NCH:
                copies[c + 1] = pltpu.async_copy(
                    table_hbm.at[pl.ds(base + (c + 1) * CHR, CHR)],
                    bufs[(c + 1) % 2], sems[(c + 1) % 2])
            copies[c].wait()
            buf = bufs[c % 2]
            cbase = base + c * CHR

            def body(r, carry):
                bv, bi = carry
                k0 = buf[r, pl.ds(0, 16)]
                k1 = buf[r, pl.ds(16, 16)]
                k2 = buf[r, pl.ds(32, 16)]
                k3 = buf[r, pl.ds(48, 16)]
                knv = _allsum(k0 * k0 + k1 * k1 + k2 * k2 + k3 * k3)
                kidx = jnp.full((NQ,), cbase + r, jnp.int32)
                for qi in range(NQ):
                    pv = (k0 * q_v[qi, pl.ds(0, 16)]
                          + k1 * q_v[qi, pl.ds(16, 16)]
                          + k2 * q_v[qi, pl.ds(32, 16)]
                          + k3 * q_v[qi, pl.ds(48, 16)])
                    dv = knv - 2.0 * _allsum(pv)
                    m = (lanes == qi) & (dv < bv)
                    bv = jnp.where(m, dv, bv)
                    bi = jnp.where(m, kidx, bi)
                return bv, bi

            best_v, best_i = lax.fori_loop(0, CHR, body, (best_v, best_i))

        resv_v[...] = best_v
        resi_v[...] = best_i
        pltpu.sync_copy(resv_v, bv_hbm.at[wid])
        pltpu.sync_copy(resi_v, bi_hbm.at[wid])

    return _sc_dist


def _merge_body(tv_ref, ti_ref, sv_ref, si_ref, q_ref, bidx_ref, minv_ref):
    v_all = jnp.concatenate([tv_ref[...], sv_ref[...]], axis=0)
    i_all = jnp.concatenate([ti_ref[...], si_ref[...]], axis=0)
    bestv = jnp.min(v_all, axis=0, keepdims=True)
    bidx = jnp.min(jnp.where(v_all == bestv, i_all, K),
                   axis=0, keepdims=True)
    bidx_ref[...] = bidx
    q = q_ref[...]
    qnt = jax.lax.dot_general(
        jnp.ones((1, D), jnp.float32), q * q, (((1,), (1,)), ((), ())),
        preferred_element_type=jnp.float32,
        precision=jax.lax.Precision.HIGHEST)
    minv_ref[...] = jnp.min(bestv + qnt).reshape(1, 1)


def _merge(tv, ti, sv, si, inpt):
    return pl.pallas_call(
        _merge_body,
        out_shape=[
            jax.ShapeDtypeStruct((1, NQ), jnp.int32),
            jax.ShapeDtypeStruct((1, 1), jnp.float32),
        ],
    )(tv, ti, sv, si, inpt)


@functools.cache
def _make_sc_gather():
    mesh = plsc.VectorSubcoreMesh(core_axis_name="c", subcore_axis_name="s")

    @functools.partial(
        pl.kernel,
        mesh=mesh,
        out_type=jax.ShapeDtypeStruct((NQ, D), jnp.float32),
        scratch_types=[
            pltpu.VMEM((NQ,), jnp.int32),
            pltpu.VMEM((NQ, D), jnp.float32),
            pltpu.SemaphoreType.DMA,
        ],
        compiler_params=pltpu.CompilerParams(use_tc_tiling_on_sc=False),
    )
    def _sc_gather(idx_hbm, table_hbm, out_hbm, idx_v, rows_v, sem):
        wid = lax.axis_index("s") * 2 + lax.axis_index("c")

        @pl.when(wid == 0)
        def _():
            pltpu.sync_copy(idx_hbm, idx_v)
            pltpu.async_copy(table_hbm.at[idx_v], rows_v, sem).wait()
            pltpu.sync_copy(rows_v, out_hbm)

    return _sc_gather


def kernel(inpt, in_memory):
    mem_packed = in_memory.reshape(K // G, DP)
    tv, ti = _tc_dist(mem_packed, inpt)
    sv, si = _make_sc_dist()(inpt, in_memory)
    bidx, minv = _merge(tv, ti, sv, si, inpt)
    matched = _make_sc_gather()(bidx.reshape(NQ), in_memory)
    return matched, minv[0, 0]
